# Initial kernel scaffold; baseline (speedup 1.0000x reference)
import functools

import jax
import jax.numpy as jnp
from jax import lax
from jax.experimental import pallas as pl
from jax.experimental.pallas import tpu as pltpu
from jax.experimental.pallas import tpu_sc as plsc

N = 10000
E = 160000
D = 256
H = 128
OUT = 256
NC = 2
NS = 16
NP = 10240
BN = 1024
EPT = E // NS
CHUNK = 80
NCHUNK = EPT // CHUNK
RPT = NP // NS

_mesh = plsc.VectorSubcoreMesh(
    core_axis_name="c", subcore_axis_name="s", num_cores=NC, num_subcores=NS)


def _sc_body(src2, dst, he, hn2, zrows, zhe, agg_out, hea_out,
             src_v, dst_v, rows_v, he_v, acc, acc_he, sem):
    cid = lax.axis_index("c")
    sid = lax.axis_index("s")

    pltpu.sync_copy(zrows, acc.at[pl.ds(sid * RPT, RPT)])

    @pl.when(cid == 1)
    def _():
        pltpu.sync_copy(zhe, acc_he.at[pl.ds(sid * RPT, RPT)])

    plsc.subcore_barrier()

    base0 = sid * EPT

    def step(c, carry):
        base = base0 + c * CHUNK
        pltpu.sync_copy(src2.at[cid, pl.ds(base, CHUNK)], src_v)
        pltpu.sync_copy(dst.at[pl.ds(base, CHUNK)], dst_v)
        pltpu.async_copy(hn2.at[src_v], rows_v, sem).wait()
        pltpu.sync_copy(rows_v, acc.at[dst_v], add=True)

        @pl.when(cid == 1)
        def _():
            pltpu.sync_copy(he.at[pl.ds(base, CHUNK)], he_v)
            pltpu.sync_copy(he_v, acc_he.at[dst_v], add=True)

        return carry

    lax.fori_loop(0, NCHUNK, step, 0)
    plsc.subcore_barrier()

    pltpu.sync_copy(acc.at[pl.ds(sid * RPT, RPT)],
                    agg_out.at[cid, pl.ds(sid * RPT, RPT)])

    @pl.when(cid == 1)
    def _():
        pltpu.sync_copy(acc_he.at[pl.ds(sid * RPT, RPT)],
                        hea_out.at[pl.ds(sid * RPT, RPT)])


_sc_call = functools.partial(
    pl.kernel,
    out_type=[
        jax.ShapeDtypeStruct((NC, NP, H), jnp.float32),
        jax.ShapeDtypeStruct((NP,), jnp.float32),
    ],
    mesh=_mesh,
    scratch_types=[
        pltpu.VMEM((CHUNK,), jnp.int32),
        pltpu.VMEM((CHUNK,), jnp.int32),
        pltpu.VMEM((CHUNK, H), jnp.float32),
        pltpu.VMEM((CHUNK,), jnp.float32),
        pltpu.VMEM_SHARED((NP, H), jnp.float32),
        pltpu.VMEM_SHARED((NP,), jnp.float32),
        pltpu.SemaphoreType.DMA,
    ],
)(_sc_body)


def _tc_body(hn_ref, agg_ref, hea_ref, w1_ref, w2_ref, wb_ref, out_ref):
    x = hn_ref[...]
    acc = jnp.dot(x, w1_ref[...], preferred_element_type=jnp.float32)
    acc += jnp.dot(agg_ref[0], w2_ref[0], preferred_element_type=jnp.float32)
    acc += jnp.dot(agg_ref[1], w2_ref[1], preferred_element_type=jnp.float32)
    hea = hea_ref[...]
    acc += hea[:, None] * wb_ref[0][None, :] + wb_ref[1][None, :]
    out_ref[...] = acc


_tc_call = pl.pallas_call(
    _tc_body,
    grid=(NP // BN,),
    in_specs=[
        pl.BlockSpec((BN, D), lambda i: (i, 0)),
        pl.BlockSpec((NC, BN, H), lambda i: (0, i, 0)),
        pl.BlockSpec((BN,), lambda i: (i,)),
        pl.BlockSpec((D, OUT), lambda i: (0, 0)),
        pl.BlockSpec((NC, H, OUT), lambda i: (0, 0, 0)),
        pl.BlockSpec((NC, OUT), lambda i: (0, 0)),
    ],
    out_specs=pl.BlockSpec((BN, OUT), lambda i: (i, 0)),
    out_shape=jax.ShapeDtypeStruct((NP, OUT), jnp.float32),
)


def kernel(hn, he, edge_index, W, b):
    src = edge_index[0].astype(jnp.int32)
    dst = edge_index[1].astype(jnp.int32)
    src2 = jnp.stack([src, src + N])
    hn2 = jnp.concatenate([hn[:, :H], hn[:, H:]], axis=0)
    he_flat = he[:, 0]
    zrows = jnp.zeros((RPT, H), jnp.float32)
    zhe = jnp.zeros((RPT,), jnp.float32)

    agg, hea = _sc_call(src2, dst, he_flat, hn2, zrows, zhe)

    hn_p = jnp.pad(hn, ((0, NP - N), (0, 0)))
    w1 = W[:D]
    w2 = jnp.stack([W[D:D + H], W[D + H:2 * D]])
    wb = jnp.stack([W[2 * D], b])

    out_p = _tc_call(hn_p, agg, hea, w1, w2, wb)
    return out_p[:N]


# speedup vs baseline: 3.4498x; 3.4498x over previous
import functools

import jax
import jax.numpy as jnp
from jax import lax
from jax.experimental import pallas as pl
from jax.experimental.pallas import tpu as pltpu
from jax.experimental.pallas import tpu_sc as plsc

N = 10000
E = 160000
D = 256
H = 128
OUT = 256
NC = 2
NS = 16
NP = 10240
BN = 1024
EPT = E // NS
CHUNK = 80
NCHUNK = EPT // CHUNK
RPT = NP // NS

def _sc_body(src2, dst, he, hn2, zrows, zhe, agg_out, hea_out,
             src_v, dst_v, rows_v, he_v, acc, acc_he, sem):
    cid = lax.axis_index("c")
    sid = lax.axis_index("s")

    pltpu.sync_copy(zrows, acc.at[pl.ds(sid * RPT, RPT)])

    @pl.when(cid == 1)
    def _():
        pltpu.sync_copy(zhe, acc_he.at[pl.ds(sid * RPT, RPT)])

    plsc.subcore_barrier()

    base0 = sid * EPT

    def step(c, carry):
        base = base0 + c * CHUNK
        pltpu.sync_copy(src2.at[pl.ds(cid * E + base, CHUNK)], src_v)
        pltpu.sync_copy(dst.at[pl.ds(base, CHUNK)], dst_v)
        pltpu.async_copy(hn2.at[src_v], rows_v, sem).wait()
        pltpu.sync_copy(rows_v, acc.at[dst_v], add=True)

        @pl.when(cid == 1)
        def _():
            pltpu.sync_copy(he.at[pl.ds(base, CHUNK)], he_v)
            pltpu.sync_copy(he_v, acc_he.at[dst_v], add=True)

        return carry

    lax.fori_loop(0, NCHUNK, step, 0)
    plsc.subcore_barrier()

    pltpu.sync_copy(acc.at[pl.ds(sid * RPT, RPT)],
                    agg_out.at[cid, pl.ds(sid * RPT, RPT)])

    @pl.when(cid == 1)
    def _():
        pltpu.sync_copy(acc_he.at[pl.ds(sid * RPT, RPT)],
                        hea_out.at[pl.ds(sid * RPT, RPT)])


@functools.lru_cache(maxsize=1)
def _make_sc_call():
    mesh = plsc.VectorSubcoreMesh(
        core_axis_name="c", subcore_axis_name="s",
        num_cores=NC, num_subcores=NS)
    return pl.kernel(
        _sc_body,
        out_type=[
            jax.ShapeDtypeStruct((NC, NP, H), jnp.float32),
            jax.ShapeDtypeStruct((NP,), jnp.float32),
        ],
        mesh=mesh,
        scratch_types=[
            pltpu.VMEM((CHUNK,), jnp.int32),
            pltpu.VMEM((CHUNK,), jnp.int32),
            pltpu.VMEM((CHUNK, H), jnp.float32),
            pltpu.VMEM((CHUNK,), jnp.float32),
            pltpu.VMEM_SHARED((NP, H), jnp.float32),
            pltpu.VMEM_SHARED((NP,), jnp.float32),
            pltpu.SemaphoreType.DMA,
        ],
    )


def _tc_body(hn_ref, agg_ref, hea_ref, w1_ref, w2_ref, wb_ref, out_ref):
    x = hn_ref[...]
    acc = jnp.dot(x, w1_ref[...], preferred_element_type=jnp.float32)
    acc += jnp.dot(agg_ref[0], w2_ref[0], preferred_element_type=jnp.float32)
    acc += jnp.dot(agg_ref[1], w2_ref[1], preferred_element_type=jnp.float32)
    hea = hea_ref[...]
    acc += hea[:, None] * wb_ref[0][None, :] + wb_ref[1][None, :]
    out_ref[...] = acc


_tc_call = pl.pallas_call(
    _tc_body,
    grid=(NP // BN,),
    in_specs=[
        pl.BlockSpec((BN, D), lambda i: (i, 0)),
        pl.BlockSpec((NC, BN, H), lambda i: (0, i, 0)),
        pl.BlockSpec((BN,), lambda i: (i,)),
        pl.BlockSpec((D, OUT), lambda i: (0, 0)),
        pl.BlockSpec((NC, H, OUT), lambda i: (0, 0, 0)),
        pl.BlockSpec((NC, OUT), lambda i: (0, 0)),
    ],
    out_specs=pl.BlockSpec((BN, OUT), lambda i: (i, 0)),
    out_shape=jax.ShapeDtypeStruct((NP, OUT), jnp.float32),
)


def kernel(hn, he, edge_index, W, b):
    src = edge_index[0].astype(jnp.int32)
    dst = edge_index[1].astype(jnp.int32)
    src2 = jnp.concatenate([src, src + N])
    hn2 = jnp.concatenate([hn[:, :H], hn[:, H:]], axis=0)
    he_flat = he[:, 0]
    zrows = jnp.zeros((RPT, H), jnp.float32)
    zhe = jnp.zeros((RPT,), jnp.float32)

    agg, hea = _make_sc_call()(src2, dst, he_flat, hn2, zrows, zhe)

    hn_p = jnp.pad(hn, ((0, NP - N), (0, 0)))
    w1 = W[:D]
    w2 = jnp.stack([W[D:D + H], W[D + H:2 * D]])
    wb = jnp.stack([W[2 * D], b])

    out_p = _tc_call(hn_p, agg, hea, w1, w2, wb)
    return out_p[:N]


# speedup vs baseline: 4.1858x; 1.2134x over previous
import functools

import jax
import jax.numpy as jnp
from jax import lax
from jax.experimental import pallas as pl
from jax.experimental.pallas import tpu as pltpu
from jax.experimental.pallas import tpu_sc as plsc

N = 10000
E = 160000
D = 256
H = 128
OUT = 256
NC = 2
NS = 16
NP = 10240
BN = 1024
EPT = E // NS
CHUNK = 128
NCH = 80
EPTP = NCH * CHUNK
SUP = 4
KPB = 2 * SUP
NBODY = NCH // KPB
RPT = NP // NS


def _sc_body(srcp, dstp, hep, hn2, zrows, zhe, agg_out, hea_out,
             src_a, dst_a, he_a, src_b, dst_b, he_b, r0, r1, acc, acc_he,
             g0, g1, s0, s1, ia, ib):
    cid = lax.axis_index("c")
    sid = lax.axis_index("s")
    rows = (r0, r1)
    gsem = (g0, g1)
    ssem = (s0, s1)
    tb_s = (cid * NS + sid) * NCH
    tb_d = sid * NCH

    def load_group(s, sbuf, dbuf, hbuf, sem):
        pltpu.async_copy(srcp.at[pl.ds(tb_s + s * SUP, SUP)], sbuf, sem)
        pltpu.async_copy(dstp.at[pl.ds(tb_d + s * SUP, SUP)], dbuf, sem)
        pltpu.async_copy(hep.at[pl.ds(tb_d + s * SUP, SUP)], hbuf, sem)

    def drain_group(sbuf, dbuf, hbuf, sem):
        pltpu.make_async_copy(srcp.at[pl.ds(0, SUP)], sbuf, sem).wait()
        pltpu.make_async_copy(dstp.at[pl.ds(0, SUP)], dbuf, sem).wait()
        pltpu.make_async_copy(hep.at[pl.ds(0, SUP)], hbuf, sem).wait()

    def wait_gather(q, src_row):
        pltpu.make_async_copy(hn2.at[src_row], rows[q], gsem[q]).wait()

    def wait_scatter(q):
        pltpu.make_async_copy(rows[q], acc.at[dst_a.at[0]], ssem[q]).wait()

    def idx_rows(k):
        if k < SUP:
            return src_a.at[k], dst_a.at[k], he_a.at[k]
        return src_b.at[k - SUP], dst_b.at[k - SUP], he_b.at[k - SUP]

    load_group(0, src_a, dst_a, he_a, ia)
    load_group(1, src_b, dst_b, he_b, ib)
    pltpu.sync_copy(zrows, acc.at[pl.ds(sid * RPT, RPT)])
    pltpu.sync_copy(zhe, acc_he.at[pl.ds(sid * RPT, RPT)])

    drain_group(src_a, dst_a, he_a, ia)
    plsc.subcore_barrier()
    pltpu.async_copy(hn2.at[src_a.at[0]], rows[0], gsem[0])

    def body(p, carry):
        for k in range(KPB):
            q = k % 2
            qn = (q + 1) % 2
            src_row, dst_row, he_row = idx_rows(k)

            wait_gather(q, src_row)
            pltpu.async_copy(rows[q], acc.at[dst_row], ssem[q], add=True)

            @pl.when(cid == (k % 2))
            def _():
                pltpu.sync_copy(he_row, acc_he.at[dst_row], add=True)

            if k == 0:
                @pl.when(p > 0)
                def _():
                    wait_scatter(qn)
                    load_group(2 * p + 1, src_b, dst_b, he_b, ib)
            else:
                wait_scatter(qn)

            if k == SUP - 1:
                drain_group(src_b, dst_b, he_b, ib)

            if k == SUP:
                @pl.when(p < NBODY - 1)
                def _():
                    load_group(2 * p + 2, src_a, dst_a, he_a, ia)

            if k < KPB - 1:
                nsrc, _, _ = idx_rows(k + 1)
                pltpu.async_copy(hn2.at[nsrc], rows[qn], gsem[qn])
            else:
                @pl.when(p < NBODY - 1)
                def _():
                    drain_group(src_a, dst_a, he_a, ia)
                    pltpu.async_copy(hn2.at[src_a.at[0]], rows[qn], gsem[qn])
        return carry

    lax.fori_loop(0, NBODY, body, 0)

    wait_scatter(1)
    plsc.subcore_barrier()

    pltpu.sync_copy(acc.at[pl.ds(sid * RPT, RPT)],
                    agg_out.at[cid, pl.ds(sid * RPT, RPT)])
    pltpu.sync_copy(acc_he.at[pl.ds(sid * RPT, RPT)],
                    hea_out.at[pl.ds(cid * NP + sid * RPT, RPT)])


@functools.lru_cache(maxsize=1)
def _make_sc_call():
    mesh = plsc.VectorSubcoreMesh(
        core_axis_name="c", subcore_axis_name="s",
        num_cores=NC, num_subcores=NS)
    return pl.kernel(
        _sc_body,
        out_type=[
            jax.ShapeDtypeStruct((NC, NP, H), jnp.float32),
            jax.ShapeDtypeStruct((NC * NP,), jnp.float32),
        ],
        mesh=mesh,
        scratch_types=[
            pltpu.VMEM((SUP, CHUNK), jnp.int32),
            pltpu.VMEM((SUP, CHUNK), jnp.int32),
            pltpu.VMEM((SUP, CHUNK), jnp.float32),
            pltpu.VMEM((SUP, CHUNK), jnp.int32),
            pltpu.VMEM((SUP, CHUNK), jnp.int32),
            pltpu.VMEM((SUP, CHUNK), jnp.float32),
            pltpu.VMEM((CHUNK, H), jnp.float32),
            pltpu.VMEM((CHUNK, H), jnp.float32),
            pltpu.VMEM_SHARED((NP, H), jnp.float32),
            pltpu.VMEM_SHARED((NP,), jnp.float32),
            pltpu.SemaphoreType.DMA,
            pltpu.SemaphoreType.DMA,
            pltpu.SemaphoreType.DMA,
            pltpu.SemaphoreType.DMA,
            pltpu.SemaphoreType.DMA,
            pltpu.SemaphoreType.DMA,
        ],
    )


def _tc_body(hn_ref, agg_ref, hea_ref, w1_ref, w2_ref, wb_ref, out_ref):
    x = hn_ref[...]
    acc = jnp.dot(x, w1_ref[...], preferred_element_type=jnp.float32)
    acc += jnp.dot(agg_ref[0], w2_ref[0], preferred_element_type=jnp.float32)
    acc += jnp.dot(agg_ref[1], w2_ref[1], preferred_element_type=jnp.float32)
    hea = hea_ref[0] + hea_ref[1]
    acc += hea[:, None] * wb_ref[0][None, :] + wb_ref[1][None, :]
    out_ref[...] = acc


_tc_call = pl.pallas_call(
    _tc_body,
    grid=(NP // BN,),
    in_specs=[
        pl.BlockSpec((BN, D), lambda i: (i, 0)),
        pl.BlockSpec((NC, BN, H), lambda i: (0, i, 0)),
        pl.BlockSpec((NC, BN), lambda i: (0, i)),
        pl.BlockSpec((D, OUT), lambda i: (0, 0)),
        pl.BlockSpec((NC, H, OUT), lambda i: (0, 0, 0)),
        pl.BlockSpec((NC, OUT), lambda i: (0, 0)),
    ],
    out_specs=pl.BlockSpec((BN, OUT), lambda i: (i, 0)),
    out_shape=jax.ShapeDtypeStruct((NP, OUT), jnp.float32),
)


def _pad_chunks(x, pad_value):
    x = x.reshape(NS, EPT)
    x = jnp.pad(x, ((0, 0), (0, EPTP - EPT)), constant_values=pad_value)
    return x.reshape(NS * NCH, CHUNK)


def kernel(hn, he, edge_index, W, b):
    src = edge_index[0].astype(jnp.int32)
    dst = edge_index[1].astype(jnp.int32)
    src_c = _pad_chunks(src, 0)
    srcp = jnp.concatenate([src_c, src_c + N])
    dstp = _pad_chunks(dst, NP - 1)
    hep = _pad_chunks(he[:, 0], 0.0)
    hn2 = jnp.concatenate([hn[:, :H], hn[:, H:]], axis=0)
    zrows = jnp.zeros((RPT, H), jnp.float32)
    zhe = jnp.zeros((RPT,), jnp.float32)

    agg, hea = _make_sc_call()(srcp, dstp, hep, hn2, zrows, zhe)
    hea2 = hea.reshape(NC, NP)

    hn_p = jnp.pad(hn, ((0, NP - N), (0, 0)))
    w1 = W[:D]
    w2 = jnp.stack([W[D:D + H], W[D + H:2 * D]])
    wb = jnp.stack([W[2 * D], b])

    out_p = _tc_call(hn_p, agg, hea2, w1, w2, wb)
    return out_p[:N]


# speedup vs baseline: 4.4888x; 1.0724x over previous
import functools

import jax
import jax.numpy as jnp
from jax import lax
from jax.experimental import pallas as pl
from jax.experimental.pallas import tpu as pltpu
from jax.experimental.pallas import tpu_sc as plsc

N = 10000
E = 160000
D = 256
H = 128
OUT = 256
NC = 2
NS = 16
NP = 10240
BN = 2000
EPT = E // NS
CHUNK = 128
NCH = 80
EPTP = NCH * CHUNK
SUP = 4
KPB = 2 * SUP
NBODY = NCH // KPB
RPT = NP // NS


def _sc_body(srcp, dstp, hep, hn2, zrows, zhe, agg_out, hea_out,
             src_a, dst_a, he_a, src_b, dst_b, he_b, r0, r1, acc, acc_he,
             g0, g1, s0, s1, ia, ib):
    cid = lax.axis_index("c")
    sid = lax.axis_index("s")
    rows = (r0, r1)
    gsem = (g0, g1)
    ssem = (s0, s1)
    tb_s = (cid * NS + sid) * NCH
    tb_d = sid * NCH

    def load_group(s, sbuf, dbuf, hbuf, sem):
        pltpu.async_copy(srcp.at[pl.ds(tb_s + s * SUP, SUP)], sbuf, sem)
        pltpu.async_copy(dstp.at[pl.ds(tb_d + s * SUP, SUP)], dbuf, sem)
        pltpu.async_copy(hep.at[pl.ds(tb_d + s * SUP, SUP)], hbuf, sem)

    def drain_group(sbuf, dbuf, hbuf, sem):
        pltpu.make_async_copy(srcp.at[pl.ds(0, SUP)], sbuf, sem).wait()
        pltpu.make_async_copy(dstp.at[pl.ds(0, SUP)], dbuf, sem).wait()
        pltpu.make_async_copy(hep.at[pl.ds(0, SUP)], hbuf, sem).wait()

    def wait_gather(q, src_row):
        pltpu.make_async_copy(hn2.at[src_row], rows[q], gsem[q]).wait()

    def wait_scatter(q):
        pltpu.make_async_copy(rows[q], acc.at[dst_a.at[0]], ssem[q]).wait()

    def idx_rows(k):
        if k < SUP:
            return src_a.at[k], dst_a.at[k], he_a.at[k]
        return src_b.at[k - SUP], dst_b.at[k - SUP], he_b.at[k - SUP]

    load_group(0, src_a, dst_a, he_a, ia)
    load_group(1, src_b, dst_b, he_b, ib)
    pltpu.sync_copy(zrows, acc.at[pl.ds(sid * RPT, RPT)])
    pltpu.sync_copy(zhe, acc_he.at[pl.ds(sid * RPT, RPT)])

    drain_group(src_a, dst_a, he_a, ia)
    plsc.subcore_barrier()
    pltpu.async_copy(hn2.at[src_a.at[0]], rows[0], gsem[0])

    def body(p, carry):
        for k in range(KPB):
            q = k % 2
            qn = (q + 1) % 2
            src_row, dst_row, he_row = idx_rows(k)

            wait_gather(q, src_row)
            pltpu.async_copy(rows[q], acc.at[dst_row], ssem[q], add=True)

            @pl.when(cid == (k % 2))
            def _():
                pltpu.sync_copy(he_row, acc_he.at[dst_row], add=True)

            if k == 0:
                @pl.when(p > 0)
                def _():
                    wait_scatter(qn)
                    load_group(2 * p + 1, src_b, dst_b, he_b, ib)
            else:
                wait_scatter(qn)

            if k == SUP - 1:
                drain_group(src_b, dst_b, he_b, ib)

            if k == SUP:
                @pl.when(p < NBODY - 1)
                def _():
                    load_group(2 * p + 2, src_a, dst_a, he_a, ia)

            if k < KPB - 1:
                nsrc, _, _ = idx_rows(k + 1)
                pltpu.async_copy(hn2.at[nsrc], rows[qn], gsem[qn])
            else:
                @pl.when(p < NBODY - 1)
                def _():
                    drain_group(src_a, dst_a, he_a, ia)
                    pltpu.async_copy(hn2.at[src_a.at[0]], rows[qn], gsem[qn])
        return carry

    lax.fori_loop(0, NBODY, body, 0)

    wait_scatter(1)
    plsc.subcore_barrier()

    pltpu.sync_copy(acc.at[pl.ds(sid * RPT, RPT)],
                    agg_out.at[cid, pl.ds(sid * RPT, RPT)])
    pltpu.sync_copy(acc_he.at[pl.ds(sid * RPT, RPT)],
                    hea_out.at[pl.ds(cid * NP + sid * RPT, RPT)])


@functools.lru_cache(maxsize=1)
def _make_sc_call():
    mesh = plsc.VectorSubcoreMesh(
        core_axis_name="c", subcore_axis_name="s",
        num_cores=NC, num_subcores=NS)
    return pl.kernel(
        _sc_body,
        out_type=[
            jax.ShapeDtypeStruct((NC, NP, H), jnp.float32),
            jax.ShapeDtypeStruct((NC * NP,), jnp.float32),
        ],
        mesh=mesh,
        scratch_types=[
            pltpu.VMEM((SUP, CHUNK), jnp.int32),
            pltpu.VMEM((SUP, CHUNK), jnp.int32),
            pltpu.VMEM((SUP, CHUNK), jnp.float32),
            pltpu.VMEM((SUP, CHUNK), jnp.int32),
            pltpu.VMEM((SUP, CHUNK), jnp.int32),
            pltpu.VMEM((SUP, CHUNK), jnp.float32),
            pltpu.VMEM((CHUNK, H), jnp.float32),
            pltpu.VMEM((CHUNK, H), jnp.float32),
            pltpu.VMEM_SHARED((NP, H), jnp.float32),
            pltpu.VMEM_SHARED((NP,), jnp.float32),
            pltpu.SemaphoreType.DMA,
            pltpu.SemaphoreType.DMA,
            pltpu.SemaphoreType.DMA,
            pltpu.SemaphoreType.DMA,
            pltpu.SemaphoreType.DMA,
            pltpu.SemaphoreType.DMA,
        ],
    )


def _tc_body(hn_ref, agg_ref, hea_ref, w1_ref, w2_ref, wb_ref, out_ref):
    x = hn_ref[...]
    acc = jnp.dot(x, w1_ref[...], preferred_element_type=jnp.float32)
    acc += jnp.dot(agg_ref[0], w2_ref[0], preferred_element_type=jnp.float32)
    acc += jnp.dot(agg_ref[1], w2_ref[1], preferred_element_type=jnp.float32)
    hea = hea_ref[:, 0] + hea_ref[:, 1]
    acc += hea[:, None] * wb_ref[0][None, :] + wb_ref[1][None, :]
    out_ref[...] = acc


_tc_call = pl.pallas_call(
    _tc_body,
    grid=(N // BN,),
    in_specs=[
        pl.BlockSpec((BN, D), lambda i: (i, 0)),
        pl.BlockSpec((NC, BN, H), lambda i: (0, i, 0)),
        pl.BlockSpec((BN, NC), lambda i: (i, 0)),
        pl.BlockSpec((D, OUT), lambda i: (0, 0)),
        pl.BlockSpec((NC, H, OUT), lambda i: (0, 0, 0)),
        pl.BlockSpec((NC, OUT), lambda i: (0, 0)),
    ],
    out_specs=pl.BlockSpec((BN, OUT), lambda i: (i, 0)),
    out_shape=jax.ShapeDtypeStruct((N, OUT), jnp.float32),
)


def _pad_chunks(x, pad_value):
    x = x.reshape(NS, EPT)
    x = jnp.pad(x, ((0, 0), (0, EPTP - EPT)), constant_values=pad_value)
    return x.reshape(NS * NCH, CHUNK)


def kernel(hn, he, edge_index, W, b):
    src = edge_index[0].astype(jnp.int32)
    dst = edge_index[1].astype(jnp.int32)
    src_c = _pad_chunks(src, 0)
    srcp = jnp.concatenate([src_c, src_c + N])
    dstp = _pad_chunks(dst, NP - 1)
    hep = _pad_chunks(he[:, 0], 0.0)
    hn2 = jnp.concatenate([hn[:, :H], hn[:, H:]], axis=0)
    zrows = jnp.zeros((RPT, H), jnp.float32)
    zhe = jnp.zeros((RPT,), jnp.float32)

    agg, hea = _make_sc_call()(srcp, dstp, hep, hn2, zrows, zhe)
    hea_t = hea.reshape(NC, NP).T

    w1 = W[:D]
    w2 = jnp.stack([W[D:D + H], W[D + H:2 * D]])
    wb = jnp.stack([W[2 * D], b])

    return _tc_call(hn, agg, hea_t, w1, w2, wb)


# speedup vs baseline: 4.5489x; 1.0134x over previous
import functools

import jax
import jax.numpy as jnp
from jax import lax
from jax.experimental import pallas as pl
from jax.experimental.pallas import tpu as pltpu
from jax.experimental.pallas import tpu_sc as plsc

N = 10000
E = 160000
D = 256
H = 128
OUT = 256
NC = 2
NS = 16
NP = 10240
BN = 2000
EPT = E // NS
CHUNK = 128
NCH = 80
EPTP = NCH * CHUNK
SUP = 4
KPB = 2 * SUP
NBODY = NCH // KPB
RPT = NP // NS


def _sc_body(srcp, dstp, hep, hn2, zrows, zhe, agg_out, hea_out,
             src_a, dst_a, he_a, src_b, dst_b, he_b, r0, r1, acc, acc_he,
             g0, g1, s0, s1, ia, ib):
    cid = lax.axis_index("c")
    sid = lax.axis_index("s")
    rows = (r0, r1)
    gsem = (g0, g1)
    ssem = (s0, s1)
    tb_s = (cid * NS + sid) * NCH
    tb_d = sid * NCH

    def load_group(s, sbuf, dbuf, hbuf, sem):
        pltpu.async_copy(srcp.at[pl.ds(tb_s + s * SUP, SUP)], sbuf, sem)
        pltpu.async_copy(dstp.at[pl.ds(tb_d + s * SUP, SUP)], dbuf, sem)
        pltpu.async_copy(hep.at[pl.ds(tb_d + s * SUP, SUP)], hbuf, sem)

    def drain_group(sbuf, dbuf, hbuf, sem):
        pltpu.make_async_copy(srcp.at[pl.ds(0, SUP)], sbuf, sem).wait()
        pltpu.make_async_copy(dstp.at[pl.ds(0, SUP)], dbuf, sem).wait()
        pltpu.make_async_copy(hep.at[pl.ds(0, SUP)], hbuf, sem).wait()

    def wait_gather(q, src_row):
        pltpu.make_async_copy(hn2.at[src_row], rows[q], gsem[q]).wait()

    def wait_scatter(q):
        pltpu.make_async_copy(rows[q], acc.at[dst_a.at[0]], ssem[q]).wait()

    def idx_rows(k):
        if k < SUP:
            return src_a.at[k], dst_a.at[k], he_a.at[k]
        return src_b.at[k - SUP], dst_b.at[k - SUP], he_b.at[k - SUP]

    load_group(0, src_a, dst_a, he_a, ia)
    load_group(1, src_b, dst_b, he_b, ib)
    pltpu.sync_copy(zrows, acc.at[pl.ds(sid * RPT, RPT)])
    pltpu.sync_copy(zhe, acc_he.at[pl.ds(sid * RPT, RPT)])

    drain_group(src_a, dst_a, he_a, ia)
    plsc.subcore_barrier()
    pltpu.async_copy(hn2.at[src_a.at[0]], rows[0], gsem[0])

    def body(p, carry):
        for k in range(KPB):
            q = k % 2
            qn = (q + 1) % 2
            src_row, dst_row, he_row = idx_rows(k)

            wait_gather(q, src_row)
            pltpu.async_copy(rows[q], acc.at[dst_row], ssem[q], add=True)

            if k == 0:
                @pl.when(p > 0)
                def _():
                    wait_scatter(qn)
                    load_group(2 * p + 1, src_b, dst_b, he_b, ib)
            else:
                wait_scatter(qn)

            if k == SUP - 1:
                drain_group(src_b, dst_b, he_b, ib)

            if k == SUP:
                @pl.when(p < NBODY - 1)
                def _():
                    load_group(2 * p + 2, src_a, dst_a, he_a, ia)

            if k < KPB - 1:
                nsrc, _, _ = idx_rows(k + 1)
                pltpu.async_copy(hn2.at[nsrc], rows[qn], gsem[qn])
            else:
                @pl.when(p < NBODY - 1)
                def _():
                    drain_group(src_a, dst_a, he_a, ia)
                    pltpu.async_copy(hn2.at[src_a.at[0]], rows[qn], gsem[qn])

            @pl.when(cid == (k % 2))
            def _():
                pltpu.sync_copy(he_row, acc_he.at[dst_row], add=True)
        return carry

    lax.fori_loop(0, NBODY, body, 0)

    wait_scatter(1)
    plsc.subcore_barrier()

    pltpu.sync_copy(acc.at[pl.ds(sid * RPT, RPT)],
                    agg_out.at[cid, pl.ds(sid * RPT, RPT)])
    pltpu.sync_copy(acc_he.at[pl.ds(sid * RPT, RPT)],
                    hea_out.at[pl.ds(cid * NP + sid * RPT, RPT)])


@functools.lru_cache(maxsize=1)
def _make_sc_call():
    mesh = plsc.VectorSubcoreMesh(
        core_axis_name="c", subcore_axis_name="s",
        num_cores=NC, num_subcores=NS)
    return pl.kernel(
        _sc_body,
        out_type=[
            jax.ShapeDtypeStruct((NC, NP, H), jnp.float32),
            jax.ShapeDtypeStruct((NC * NP,), jnp.float32),
        ],
        mesh=mesh,
        scratch_types=[
            pltpu.VMEM((SUP, CHUNK), jnp.int32),
            pltpu.VMEM((SUP, CHUNK), jnp.int32),
            pltpu.VMEM((SUP, CHUNK), jnp.float32),
            pltpu.VMEM((SUP, CHUNK), jnp.int32),
            pltpu.VMEM((SUP, CHUNK), jnp.int32),
            pltpu.VMEM((SUP, CHUNK), jnp.float32),
            pltpu.VMEM((CHUNK, H), jnp.float32),
            pltpu.VMEM((CHUNK, H), jnp.float32),
            pltpu.VMEM_SHARED((NP, H), jnp.float32),
            pltpu.VMEM_SHARED((NP,), jnp.float32),
            pltpu.SemaphoreType.DMA,
            pltpu.SemaphoreType.DMA,
            pltpu.SemaphoreType.DMA,
            pltpu.SemaphoreType.DMA,
            pltpu.SemaphoreType.DMA,
            pltpu.SemaphoreType.DMA,
        ],
    )


def _tc_body(hn_ref, agg_ref, hea_ref, w1_ref, w2_ref, wb_ref, out_ref):
    x = hn_ref[...]
    acc = jnp.dot(x, w1_ref[...], preferred_element_type=jnp.float32)
    acc += jnp.dot(agg_ref[0], w2_ref[0], preferred_element_type=jnp.float32)
    acc += jnp.dot(agg_ref[1], w2_ref[1], preferred_element_type=jnp.float32)
    hea = hea_ref[:, 0] + hea_ref[:, 1]
    acc += hea[:, None] * wb_ref[0][None, :] + wb_ref[1][None, :]
    out_ref[...] = acc


_tc_call = pl.pallas_call(
    _tc_body,
    grid=(N // BN,),
    in_specs=[
        pl.BlockSpec((BN, D), lambda i: (i, 0)),
        pl.BlockSpec((NC, BN, H), lambda i: (0, i, 0)),
        pl.BlockSpec((BN, NC), lambda i: (i, 0)),
        pl.BlockSpec((D, OUT), lambda i: (0, 0)),
        pl.BlockSpec((NC, H, OUT), lambda i: (0, 0, 0)),
        pl.BlockSpec((NC, OUT), lambda i: (0, 0)),
    ],
    out_specs=pl.BlockSpec((BN, OUT), lambda i: (i, 0)),
    out_shape=jax.ShapeDtypeStruct((N, OUT), jnp.float32),
)


def _pad_chunks(x, pad_value):
    x = x.reshape(NS, EPT)
    x = jnp.pad(x, ((0, 0), (0, EPTP - EPT)), constant_values=pad_value)
    return x.reshape(NS * NCH, CHUNK)


def kernel(hn, he, edge_index, W, b):
    src = edge_index[0].astype(jnp.int32)
    dst = edge_index[1].astype(jnp.int32)
    src_c = _pad_chunks(src, 0)
    srcp = jnp.concatenate([src_c, src_c + N])
    dstp = _pad_chunks(dst, NP - 1)
    hep = _pad_chunks(he[:, 0], 0.0)
    hn2 = jnp.concatenate([hn[:, :H], hn[:, H:]], axis=0)
    zrows = jnp.zeros((RPT, H), jnp.float32)
    zhe = jnp.zeros((RPT,), jnp.float32)

    agg, hea = _make_sc_call()(srcp, dstp, hep, hn2, zrows, zhe)
    hea_t = hea.reshape(NC, NP).T

    w1 = W[:D]
    w2 = jnp.stack([W[D:D + H], W[D + H:2 * D]])
    wb = jnp.stack([W[2 * D], b])

    return _tc_call(hn, agg, hea_t, w1, w2, wb)
